# baseline (speedup 1.0000x reference)
import functools

import jax
import jax.numpy as jnp
from jax import lax
from jax.experimental import pallas as pl
from jax.experimental.pallas import tpu as pltpu
from jax.experimental.pallas import tpu_sc as plsc

R = 4
C_OUT = 2
NC = 2
NS = 16
NW = NC * NS
LANES = 16
CH = 6400


def _sc_gtconv(w, Wb):
    E = w.shape[1]
    nchunks = E // CH
    full_rounds = nchunks // NW
    tail = nchunks - full_rounds * NW
    mesh = plsc.VectorSubcoreMesh(
        core_axis_name="c", subcore_axis_name="s",
        num_cores=NC, num_subcores=NS,
    )

    @functools.partial(
        pl.kernel,
        out_type=jax.ShapeDtypeStruct((C_OUT, E), jnp.float32),
        mesh=mesh,
        scratch_types=[
            pltpu.VMEM((2, R, CH), jnp.float32),
            pltpu.VMEM((2, C_OUT, CH), jnp.float32),
            pltpu.VMEM((C_OUT * R, LANES), jnp.float32),
            pltpu.SemaphoreType.DMA,
            pltpu.SemaphoreType.DMA,
            pltpu.SemaphoreType.DMA,
            pltpu.SemaphoreType.DMA,
        ],
    )
    def sck(w_hbm, wb_hbm, out_hbm, wbuf, obuf, fbuf,
            isem0, isem1, osem0, osem1):
        isems = (isem0, isem1)
        osems = (osem0, osem1)
        wid = lax.axis_index("s") * NC + lax.axis_index("c")
        pltpu.sync_copy(wb_hbm, fbuf)
        v = [fbuf[q] for q in range(C_OUT * R)]
        fvec = []
        for c in range(C_OUT):
            vr = v[c * R:(c + 1) * R]
            m = jnp.maximum(jnp.maximum(vr[0], vr[1]), jnp.maximum(vr[2], vr[3]))
            e = [jnp.exp(x - m) for x in vr]
            s = (e[0] + e[1]) + (e[2] + e[3])
            fvec.extend([x / s for x in e])

        def offset(t):
            return (t * NW + wid) * CH

        def start_in(t, b):
            return pltpu.async_copy(
                w_hbm.at[:, pl.ds(offset(t), CH)], wbuf.at[b], isems[b]
            )

        def start_out(t, b):
            return pltpu.async_copy(
                obuf.at[b], out_hbm.at[:, pl.ds(offset(t), CH)], osems[b]
            )

        def compute(b):
            wb = wbuf.at[b]
            ob = obuf.at[b]

            def body(i, carry):
                s = pl.ds(i * LANES, LANES)
                w0 = wb[0, s]
                w1 = wb[1, s]
                w2 = wb[2, s]
                w3 = wb[3, s]
                ob[0, s] = fvec[0] * w0 + fvec[1] * w1 + fvec[2] * w2 + fvec[3] * w3
                ob[1, s] = fvec[4] * w0 + fvec[5] * w1 + fvec[6] * w2 + fvec[7] * w3
                return carry

            lax.fori_loop(0, CH // LANES, body, 0, unroll=8)

        h_in = [None, None]
        h_out = [None, None]
        h_in[0] = start_in(0, 0)
        for t in range(full_rounds):
            b = t & 1
            if t + 1 < full_rounds:
                h_in[1 - b] = start_in(t + 1, 1 - b)
            h_in[b].wait()
            if h_out[b] is not None:
                h_out[b].wait()
            compute(b)
            h_out[b] = start_out(t, b)
        h_out[0].wait()
        h_out[1].wait()

        if tail:
            @pl.when(wid < tail)
            def _():
                h = start_in(full_rounds, 0)
                h.wait()
                compute(0)
                ho = start_out(full_rounds, 0)
                ho.wait()

    return sck(w, Wb)


def _softmax_body(W_ref, filt_ref):
    Wv = W_ref[...]
    m = jnp.max(Wv, axis=1, keepdims=True)
    e = jnp.exp(Wv - m)
    filt_ref[...] = e / jnp.sum(e, axis=1, keepdims=True)


def _softmax_filter(W):
    return pl.pallas_call(
        _softmax_body,
        out_shape=jax.ShapeDtypeStruct((C_OUT, R), jnp.float32),
    )(W)


def kernel(w, edge_index, W):
    del edge_index
    Wb = jnp.broadcast_to(jnp.reshape(W, (C_OUT * R, 1)), (C_OUT * R, LANES))
    w_sum = _sc_gtconv(w, Wb)
    Filter = _softmax_filter(W)
    return (w_sum, Filter)


# speedup vs baseline: 1.7817x; 1.7817x over previous
import functools

import jax
import jax.numpy as jnp
from jax import lax
from jax.experimental import pallas as pl
from jax.experimental.pallas import tpu as pltpu
from jax.experimental.pallas import tpu_sc as plsc

R = 4
C_OUT = 2
NC = 2
NS = 16
NW = NC * NS
LANES = 16
CH = 6400


def _sc_gtconv(w, Wb):
    E = w.shape[1]
    nchunks = E // CH
    full_rounds = nchunks // NW
    tail = nchunks - full_rounds * NW
    mesh = plsc.VectorSubcoreMesh(
        core_axis_name="c", subcore_axis_name="s",
        num_cores=NC, num_subcores=NS,
    )

    @functools.partial(
        pl.kernel,
        out_type=jax.ShapeDtypeStruct((C_OUT, E), jnp.float32),
        mesh=mesh,
        scratch_types=[
            pltpu.VMEM((2, R, CH), jnp.float32),
            pltpu.VMEM((2, C_OUT, CH), jnp.float32),
            pltpu.VMEM((C_OUT * R, LANES), jnp.float32),
            pltpu.SemaphoreType.DMA,
            pltpu.SemaphoreType.DMA,
            pltpu.SemaphoreType.DMA,
            pltpu.SemaphoreType.DMA,
        ],
    )
    def sck(w_hbm, wb_hbm, out_hbm, wbuf, obuf, fbuf,
            isem0, isem1, osem0, osem1):
        isems = (isem0, isem1)
        osems = (osem0, osem1)
        wid = lax.axis_index("s") * NC + lax.axis_index("c")
        pltpu.sync_copy(wb_hbm, fbuf)
        v = [fbuf[q] for q in range(C_OUT * R)]
        fvec = []
        for c in range(C_OUT):
            vr = v[c * R:(c + 1) * R]
            m = jnp.maximum(jnp.maximum(vr[0], vr[1]), jnp.maximum(vr[2], vr[3]))
            e = [jnp.exp(x - m) for x in vr]
            s = (e[0] + e[1]) + (e[2] + e[3])
            fvec.extend([x / s for x in e])

        def offset(t):
            return (t * NW + wid) * CH

        def start_in(t, b):
            return pltpu.async_copy(
                w_hbm.at[:, pl.ds(offset(t), CH)], wbuf.at[b], isems[b]
            )

        def start_out(t, b):
            return pltpu.async_copy(
                obuf.at[b], out_hbm.at[:, pl.ds(offset(t), CH)], osems[b]
            )

        def compute(b):
            wb = wbuf.at[b]
            ob = obuf.at[b]

            @plsc.parallel_loop(0, CH // LANES, unroll=8)
            def body(i):
                s = pl.ds(i * LANES, LANES)
                w0 = wb[0, s]
                w1 = wb[1, s]
                w2 = wb[2, s]
                w3 = wb[3, s]
                ob[0, s] = fvec[0] * w0 + fvec[1] * w1 + fvec[2] * w2 + fvec[3] * w3
                ob[1, s] = fvec[4] * w0 + fvec[5] * w1 + fvec[6] * w2 + fvec[7] * w3

        h_in = [None, None]
        h_out = [None, None]
        h_in[0] = start_in(0, 0)
        for t in range(full_rounds):
            b = t & 1
            if t + 1 < full_rounds:
                h_in[1 - b] = start_in(t + 1, 1 - b)
            h_in[b].wait()
            if h_out[b] is not None:
                h_out[b].wait()
            compute(b)
            h_out[b] = start_out(t, b)
        h_out[0].wait()
        h_out[1].wait()

        if tail:
            @pl.when(wid < tail)
            def _():
                h = start_in(full_rounds, 0)
                h.wait()
                compute(0)
                ho = start_out(full_rounds, 0)
                ho.wait()

    return sck(w, Wb)


def _softmax_body(W_ref, filt_ref):
    Wv = W_ref[...]
    m = jnp.max(Wv, axis=1, keepdims=True)
    e = jnp.exp(Wv - m)
    filt_ref[...] = e / jnp.sum(e, axis=1, keepdims=True)


def _softmax_filter(W):
    return pl.pallas_call(
        _softmax_body,
        out_shape=jax.ShapeDtypeStruct((C_OUT, R), jnp.float32),
    )(W)


def kernel(w, edge_index, W):
    del edge_index
    Wb = jnp.broadcast_to(jnp.reshape(W, (C_OUT * R, 1)), (C_OUT * R, LANES))
    w_sum = _sc_gtconv(w, Wb)
    Filter = _softmax_filter(W)
    return (w_sum, Filter)


# speedup vs baseline: 1.9405x; 1.0891x over previous
import functools

import jax
import jax.numpy as jnp
from jax import lax
from jax.experimental import pallas as pl
from jax.experimental.pallas import tpu as pltpu
from jax.experimental.pallas import tpu_sc as plsc

R = 4
C_OUT = 2
NC = 2
NS = 16
NW = NC * NS
LANES = 16
CH = 6400
BLOCK_E = 800000
E_TC = 4800000


def _tc_body(W_ref, w_ref, out_ref, filt_ref):
    Wv = W_ref[...]
    m = jnp.max(Wv, axis=1, keepdims=True)
    e = jnp.exp(Wv - m)
    f = e / jnp.sum(e, axis=1, keepdims=True)
    filt_ref[...] = f
    wb = w_ref[...]
    out_ref[...] = jax.lax.dot_general(
        f, wb, (((1,), (0,)), ((), ())), preferred_element_type=jnp.float32
    )


def _tc_stream(w, W, E):
    return pl.pallas_call(
        _tc_body,
        grid=(E_TC // BLOCK_E,),
        in_specs=[
            pl.BlockSpec((C_OUT, R), lambda i: (0, 0)),
            pl.BlockSpec((R, BLOCK_E), lambda i: (0, i)),
        ],
        out_specs=[
            pl.BlockSpec((C_OUT, BLOCK_E), lambda i: (0, i)),
            pl.BlockSpec((C_OUT, R), lambda i: (0, 0)),
        ],
        out_shape=[
            jax.ShapeDtypeStruct((C_OUT, E), jnp.float32),
            jax.ShapeDtypeStruct((C_OUT, R), jnp.float32),
        ],
    )(W, w)


def _sc_stream(w, Wb):
    E = w.shape[1]
    E_SC = E - E_TC
    nchunks = E_SC // CH
    full_rounds = nchunks // NW
    tail = nchunks - full_rounds * NW
    mesh = plsc.VectorSubcoreMesh(
        core_axis_name="c", subcore_axis_name="s",
        num_cores=NC, num_subcores=NS,
    )

    @functools.partial(
        pl.kernel,
        out_type=jax.ShapeDtypeStruct((C_OUT, E_SC), jnp.float32),
        mesh=mesh,
        scratch_types=[
            pltpu.VMEM((2, R, CH), jnp.float32),
            pltpu.VMEM((2, C_OUT, CH), jnp.float32),
            pltpu.VMEM((C_OUT * R, LANES), jnp.float32),
            pltpu.SemaphoreType.DMA,
            pltpu.SemaphoreType.DMA,
            pltpu.SemaphoreType.DMA,
            pltpu.SemaphoreType.DMA,
        ],
    )
    def sck(w_hbm, wb_hbm, out_hbm, wbuf, obuf, fbuf,
            isem0, isem1, osem0, osem1):
        isems = (isem0, isem1)
        osems = (osem0, osem1)
        wid = lax.axis_index("s") * NC + lax.axis_index("c")
        pltpu.sync_copy(wb_hbm, fbuf)
        v = [fbuf[q] for q in range(C_OUT * R)]
        fvec = []
        for c in range(C_OUT):
            vr = v[c * R:(c + 1) * R]
            m = jnp.maximum(jnp.maximum(vr[0], vr[1]), jnp.maximum(vr[2], vr[3]))
            e = [jnp.exp(x - m) for x in vr]
            s = (e[0] + e[1]) + (e[2] + e[3])
            fvec.extend([x / s for x in e])

        def offset(t):
            return (t * NW + wid) * CH

        def start_in(t, b):
            return pltpu.async_copy(
                w_hbm.at[:, pl.ds(E_TC + offset(t), CH)], wbuf.at[b], isems[b]
            )

        def start_out(t, b):
            return pltpu.async_copy(
                obuf.at[b], out_hbm.at[:, pl.ds(offset(t), CH)], osems[b]
            )

        def compute(b):
            wb = wbuf.at[b]
            ob = obuf.at[b]

            @plsc.parallel_loop(0, CH // LANES, unroll=8)
            def body(i):
                s = pl.ds(i * LANES, LANES)
                w0 = wb[0, s]
                w1 = wb[1, s]
                w2 = wb[2, s]
                w3 = wb[3, s]
                ob[0, s] = fvec[0] * w0 + fvec[1] * w1 + fvec[2] * w2 + fvec[3] * w3
                ob[1, s] = fvec[4] * w0 + fvec[5] * w1 + fvec[6] * w2 + fvec[7] * w3

        h_in = [None, None]
        h_out = [None, None]
        h_in[0] = start_in(0, 0)
        for t in range(full_rounds):
            b = t & 1
            if t + 1 < full_rounds:
                h_in[1 - b] = start_in(t + 1, 1 - b)
            h_in[b].wait()
            if h_out[b] is not None:
                h_out[b].wait()
            compute(b)
            h_out[b] = start_out(t, b)
        h_out[0].wait()
        h_out[1].wait()

        if tail:
            @pl.when(wid < tail)
            def _():
                h = start_in(full_rounds, 0)
                h.wait()
                compute(0)
                ho = start_out(full_rounds, 0)
                ho.wait()

    return sck(w, Wb)


def kernel(w, edge_index, W):
    del edge_index
    E = w.shape[1]
    Wb = jnp.broadcast_to(jnp.reshape(W, (C_OUT * R, 1)), (C_OUT * R, LANES))
    sc_part = _sc_stream(w, Wb)
    w_full, Filter = _tc_stream(w, W, E)
    w_sum = lax.dynamic_update_slice(w_full, sc_part, (0, E_TC))
    return (w_sum, Filter)
